# Initial kernel scaffold; baseline (speedup 1.0000x reference)
import jax
import jax.numpy as jnp
from jax.experimental import pallas as pl


def kernel(act_flag, feat, edge_index, ntype, W_self, b_self, W_neigh, b_neigh):
    raise NotImplementedError("write your pallas kernel here")


# speedup vs baseline: 5.8761x; 5.8761x over previous
import functools

import jax
import jax.numpy as jnp
from jax import lax
from jax.experimental import pallas as pl
from jax.experimental.pallas import tpu as pltpu
from jax.experimental.pallas import tpu_sc as plsc

N = 10000
E = 320000
D = 128
NT = 8
OUT = 128

NC = 2
NS = 16
DH = D // NC
EPT = E // NS
CH = 80
NIB = 2
NCHUNK = EPT // (CH * NIB)
NPAD = 10240
ROWS_PT = NPAD // NS
ZR = 128
DEGW = 16
L = 16


def _sc_aggregate(feat2, src3d, dst3d):
  mesh = plsc.VectorSubcoreMesh(core_axis_name="c", subcore_axis_name="s")

  @functools.partial(
      pl.kernel,
      mesh=mesh,
      out_type=[
          jax.ShapeDtypeStruct((NC, NPAD, DH), jnp.float32),
          jax.ShapeDtypeStruct((NC, NPAD, DEGW), jnp.float32),
      ],
      scratch_types=[
          pltpu.VMEM((NCHUNK, CH), jnp.int32),
          pltpu.VMEM((NCHUNK, CH), jnp.int32),
          pltpu.VMEM((CH, DH), jnp.float32),
          pltpu.VMEM((CH, DEGW), jnp.float32),
          pltpu.VMEM((ZR, DH), jnp.float32),
          pltpu.VMEM((ZR, DEGW), jnp.float32),
          pltpu.VMEM_SHARED((NPAD, DH), jnp.float32),
          pltpu.VMEM_SHARED((NPAD, DEGW), jnp.float32),
          pltpu.SemaphoreType.DMA,
      ],
      compiler_params=pltpu.CompilerParams(use_tc_tiling_on_sc=False),
  )
  def agg(feat_hbm, src_hbm, dst_hbm, psum_hbm, pdeg_hbm,
          src_v, dst_v, rows_v, ones_v, zb_v, zbd_v, acc_sum, acc_deg, sem):
    c = lax.axis_index("c")
    s = lax.axis_index("s")

    zeros = jnp.zeros((L,), jnp.float32)
    ones = jnp.full((L,), 1.0, jnp.float32)

    def fill_bufs(i, _):
      for j in range(DH // L):
        zb_v[i, pl.ds(j * L, L)] = zeros
      zbd_v[i] = zeros
      return 0
    lax.fori_loop(0, ZR, fill_bufs, 0)

    def fill_ones(i, _):
      ones_v[i] = ones
      return 0
    lax.fori_loop(0, CH, fill_ones, 0)

    for r in range(ROWS_PT // ZR):
      off = s * ROWS_PT + r * ZR
      pltpu.sync_copy(zb_v, acc_sum.at[pl.ds(off, ZR)])
      pltpu.sync_copy(zbd_v, acc_deg.at[pl.ds(off, ZR)])
    plsc.subcore_barrier()

    for p in range(NIB):
      blk = s * NIB + p
      pltpu.sync_copy(src_hbm.at[blk], src_v)
      pltpu.sync_copy(dst_hbm.at[blk], dst_v)

      def step(j, _):
        pltpu.async_copy(feat_hbm.at[c].at[src_v.at[j]], rows_v, sem).wait()
        pltpu.sync_copy(rows_v, acc_sum.at[dst_v.at[j]], add=True)
        pltpu.sync_copy(ones_v, acc_deg.at[dst_v.at[j]], add=True)
        return 0
      lax.fori_loop(0, NCHUNK, step, 0)

    plsc.subcore_barrier()
    for r in range(ROWS_PT // ZR):
      off = s * ROWS_PT + r * ZR
      pltpu.sync_copy(acc_sum.at[pl.ds(off, ZR)], psum_hbm.at[c, pl.ds(off, ZR)])
      pltpu.sync_copy(acc_deg.at[pl.ds(off, ZR)], pdeg_hbm.at[c, pl.ds(off, ZR)])

  return agg(feat2, src3d, dst3d)


BLK = 1024


def _combine_body(psum_ref, pdeg_ref, ntype_ref, ws_ref, bs_ref, wn_ref,
                  bn_ref, out_ref):
  s = jnp.concatenate([psum_ref[0], psum_ref[1]], axis=-1)
  deg = 0.5 * (jnp.max(pdeg_ref[0], axis=-1) + jnp.max(pdeg_ref[1], axis=-1))
  h = s / jnp.maximum(deg, 1.0)[:, None]
  self_part = lax.dot_general(
      ntype_ref[...], ws_ref[...], (((1,), (1,)), ((), ())),
      preferred_element_type=jnp.float32)
  neigh_part = lax.dot_general(
      h, wn_ref[...], (((1,), (1,)), ((), ())),
      preferred_element_type=jnp.float32)
  out_ref[...] = self_part + neigh_part + bs_ref[...] + bn_ref[...]


def _combine(psum, pdeg, ntype, W_self, b_self, W_neigh, b_neigh):
  return pl.pallas_call(
      _combine_body,
      grid=(NPAD // BLK,),
      in_specs=[
          pl.BlockSpec((NC, BLK, DH), lambda i: (0, i, 0)),
          pl.BlockSpec((NC, BLK, DEGW), lambda i: (0, i, 0)),
          pl.BlockSpec((BLK, NT), lambda i: (i, 0)),
          pl.BlockSpec((OUT, NT), lambda i: (0, 0)),
          pl.BlockSpec((1, OUT), lambda i: (0, 0)),
          pl.BlockSpec((OUT, D), lambda i: (0, 0)),
          pl.BlockSpec((1, OUT), lambda i: (0, 0)),
      ],
      out_specs=pl.BlockSpec((BLK, OUT), lambda i: (i, 0)),
      out_shape=jax.ShapeDtypeStruct((NPAD, OUT), jnp.float32),
  )(psum, pdeg, ntype, W_self, b_self, W_neigh, b_neigh)


def kernel(act_flag, feat, edge_index, ntype, W_self, b_self, W_neigh,
           b_neigh):
  del act_flag
  feat2 = jnp.stack([feat[:, :DH], feat[:, DH:]])
  src3d = edge_index[0].reshape(NS * NIB, NCHUNK, CH)
  dst3d = edge_index[1].reshape(NS * NIB, NCHUNK, CH)
  psum, pdeg = _sc_aggregate(feat2, src3d, dst3d)
  ntype_pad = jnp.pad(ntype, ((0, NPAD - N), (0, 0)))
  rst = _combine(psum, pdeg, ntype_pad, W_self, b_self.reshape(1, OUT),
                 W_neigh, b_neigh.reshape(1, OUT))
  return rst[:N]


# speedup vs baseline: 9.1367x; 1.5549x over previous
import functools

import jax
import jax.numpy as jnp
from jax import lax
from jax.experimental import pallas as pl
from jax.experimental.pallas import tpu as pltpu
from jax.experimental.pallas import tpu_sc as plsc

N = 10000
E = 320000
D = 128
NT = 8
OUT = 128

NC = 2
NS = 16
DH = D // NC
EPT = E // NS
CH = 100
NIB = 2
NCHUNK = EPT // (CH * NIB)
NPAD = 10240
ROWS_PT = NPAD // NS
ZR = 128
DEGW = 16
L = 16


def _sc_aggregate(feat2, src3d, dst3d):
  mesh = plsc.VectorSubcoreMesh(core_axis_name="c", subcore_axis_name="s")

  @functools.partial(
      pl.kernel,
      mesh=mesh,
      out_type=[
          jax.ShapeDtypeStruct((NC, NPAD, DH), jnp.float32),
          jax.ShapeDtypeStruct((NC, NPAD, DEGW), jnp.float32),
      ],
      scratch_types=[
          pltpu.VMEM((NCHUNK, CH), jnp.int32),
          pltpu.VMEM((NCHUNK, CH), jnp.int32),
          pltpu.VMEM((CH, DH), jnp.float32),
          pltpu.VMEM((CH, DH), jnp.float32),
          pltpu.VMEM((CH, DEGW), jnp.float32),
          pltpu.VMEM((ZR, DH), jnp.float32),
          pltpu.VMEM((ZR, DEGW), jnp.float32),
          pltpu.VMEM_SHARED((NPAD, DH), jnp.float32),
          pltpu.VMEM_SHARED((NPAD, DEGW), jnp.float32),
          pltpu.SemaphoreType.DMA,
          pltpu.SemaphoreType.DMA,
      ],
      compiler_params=pltpu.CompilerParams(use_tc_tiling_on_sc=False),
  )
  def agg(feat_hbm, src_hbm, dst_hbm, psum_hbm, pdeg_hbm,
          src_v, dst_v, rows0_v, rows1_v, ones_v, zb_v, zbd_v,
          acc_sum, acc_deg, sem0, sem1):
    c = lax.axis_index("c")
    s = lax.axis_index("s")

    zeros = jnp.zeros((L,), jnp.float32)
    ones = jnp.full((L,), 1.0, jnp.float32)

    def fill_bufs(i, _):
      for j in range(DH // L):
        zb_v[i, pl.ds(j * L, L)] = zeros
      zbd_v[i] = zeros
      return 0
    lax.fori_loop(0, ZR, fill_bufs, 0)

    def fill_ones(i, _):
      ones_v[i] = ones
      return 0
    lax.fori_loop(0, CH, fill_ones, 0)

    for r in range(ROWS_PT // ZR):
      off = s * ROWS_PT + r * ZR
      pltpu.sync_copy(zb_v, acc_sum.at[pl.ds(off, ZR)])
      pltpu.sync_copy(zbd_v, acc_deg.at[pl.ds(off, ZR)])
    plsc.subcore_barrier()

    def fire(j, buf, sem):
      pltpu.async_copy(feat_hbm.at[c].at[src_v.at[j]], buf, sem)

    def drain_scatter(j, buf, sem):
      pltpu.make_async_copy(feat_hbm.at[c].at[src_v.at[j]], buf, sem).wait()
      pltpu.sync_copy(buf, acc_sum.at[dst_v.at[j]], add=True)
      pltpu.sync_copy(ones_v, acc_deg.at[dst_v.at[j]], add=True)

    for p in range(NIB):
      blk = s * NIB + p
      pltpu.sync_copy(src_hbm.at[blk], src_v)
      pltpu.sync_copy(dst_hbm.at[blk], dst_v)

      fire(0, rows0_v, sem0)

      def step(j2, _):
        j = 2 * j2
        fire(j + 1, rows1_v, sem1)
        drain_scatter(j, rows0_v, sem0)

        @pl.when(j2 < NCHUNK // 2 - 1)
        def _():
          fire(j + 2, rows0_v, sem0)
        drain_scatter(j + 1, rows1_v, sem1)
        return 0
      lax.fori_loop(0, NCHUNK // 2, step, 0)

    plsc.subcore_barrier()
    for r in range(ROWS_PT // ZR):
      off = s * ROWS_PT + r * ZR
      pltpu.sync_copy(acc_sum.at[pl.ds(off, ZR)], psum_hbm.at[c, pl.ds(off, ZR)])
      pltpu.sync_copy(acc_deg.at[pl.ds(off, ZR)], pdeg_hbm.at[c, pl.ds(off, ZR)])

  return agg(feat2, src3d, dst3d)


BLK = 1024


def _combine_body(psum_ref, pdeg_ref, ntype_ref, ws_ref, bs_ref, wn_ref,
                  bn_ref, out_ref):
  s = jnp.concatenate([psum_ref[0], psum_ref[1]], axis=-1)
  deg = 0.5 * (jnp.max(pdeg_ref[0], axis=-1) + jnp.max(pdeg_ref[1], axis=-1))
  h = s / jnp.maximum(deg, 1.0)[:, None]
  self_part = lax.dot_general(
      ntype_ref[...], ws_ref[...], (((1,), (1,)), ((), ())),
      preferred_element_type=jnp.float32)
  neigh_part = lax.dot_general(
      h, wn_ref[...], (((1,), (1,)), ((), ())),
      preferred_element_type=jnp.float32)
  out_ref[...] = self_part + neigh_part + bs_ref[...] + bn_ref[...]


def _combine(psum, pdeg, ntype, W_self, b_self, W_neigh, b_neigh):
  return pl.pallas_call(
      _combine_body,
      grid=(NPAD // BLK,),
      in_specs=[
          pl.BlockSpec((NC, BLK, DH), lambda i: (0, i, 0)),
          pl.BlockSpec((NC, BLK, DEGW), lambda i: (0, i, 0)),
          pl.BlockSpec((BLK, NT), lambda i: (i, 0)),
          pl.BlockSpec((OUT, NT), lambda i: (0, 0)),
          pl.BlockSpec((1, OUT), lambda i: (0, 0)),
          pl.BlockSpec((OUT, D), lambda i: (0, 0)),
          pl.BlockSpec((1, OUT), lambda i: (0, 0)),
      ],
      out_specs=pl.BlockSpec((BLK, OUT), lambda i: (i, 0)),
      out_shape=jax.ShapeDtypeStruct((NPAD, OUT), jnp.float32),
  )(psum, pdeg, ntype, W_self, b_self, W_neigh, b_neigh)


def kernel(act_flag, feat, edge_index, ntype, W_self, b_self, W_neigh,
           b_neigh):
  del act_flag
  feat2 = jnp.stack([feat[:, :DH], feat[:, DH:]])
  src3d = edge_index[0].reshape(NS * NIB, NCHUNK, CH)
  dst3d = edge_index[1].reshape(NS * NIB, NCHUNK, CH)
  psum, pdeg = _sc_aggregate(feat2, src3d, dst3d)
  ntype_pad = jnp.pad(ntype, ((0, NPAD - N), (0, 0)))
  rst = _combine(psum, pdeg, ntype_pad, W_self, b_self.reshape(1, OUT),
                 W_neigh, b_neigh.reshape(1, OUT))
  return rst[:N]


# speedup vs baseline: 10.5968x; 1.1598x over previous
import functools

import jax
import jax.numpy as jnp
from jax import lax
from jax.experimental import pallas as pl
from jax.experimental.pallas import tpu as pltpu
from jax.experimental.pallas import tpu_sc as plsc

N = 10000
E = 320000
D = 128
NT = 8
OUT = 128

NC = 2
NS = 16
DH = D // NC
EPT = E // NS
CH = 125
NIB = 2
NCHUNK = EPT // (CH * NIB)
NPAD = 10240
ROWS_PT = NPAD // NS
ZR = 128
DEGW = 16
L = 16


def _sc_aggregate(feat2, edge4d):
  mesh = plsc.VectorSubcoreMesh(core_axis_name="c", subcore_axis_name="s")

  @functools.partial(
      pl.kernel,
      mesh=mesh,
      out_type=[
          jax.ShapeDtypeStruct((NC, NPAD, DH), jnp.float32),
          jax.ShapeDtypeStruct((NC, NPAD, DEGW), jnp.float32),
      ],
      scratch_types=[
          pltpu.VMEM((NCHUNK, CH), jnp.int32),
          pltpu.VMEM((NCHUNK, CH), jnp.int32),
          pltpu.VMEM((CH, DH), jnp.float32),
          pltpu.VMEM((CH, DH), jnp.float32),
          pltpu.VMEM((CH, DEGW), jnp.float32),
          pltpu.VMEM((ZR, DH), jnp.float32),
          pltpu.VMEM((ZR, DEGW), jnp.float32),
          pltpu.VMEM_SHARED((NPAD, DH), jnp.float32),
          pltpu.VMEM_SHARED((NPAD, DEGW), jnp.float32),
          pltpu.SemaphoreType.DMA,
          pltpu.SemaphoreType.DMA,
      ],
      compiler_params=pltpu.CompilerParams(use_tc_tiling_on_sc=False),
  )
  def agg(feat_hbm, edge_hbm, psum_hbm, pdeg_hbm,
          src_v, dst_v, rows0_v, rows1_v, ones_v, zb_v, zbd_v,
          acc_sum, acc_deg, sem0, sem1):
    c = lax.axis_index("c")
    s = lax.axis_index("s")

    zeros = jnp.zeros((L,), jnp.float32)
    ones = jnp.full((L,), 1.0, jnp.float32)

    def fill_bufs(i, _):
      for j in range(DH // L):
        zb_v[i, pl.ds(j * L, L)] = zeros
      zbd_v[i] = zeros
      return 0
    lax.fori_loop(0, ZR, fill_bufs, 0)

    def fill_ones(i, _):
      ones_v[i] = ones
      return 0
    lax.fori_loop(0, CH, fill_ones, 0)

    for r in range(ROWS_PT // ZR):
      off = s * ROWS_PT + r * ZR
      pltpu.sync_copy(zb_v, acc_sum.at[pl.ds(off, ZR)])
      pltpu.sync_copy(zbd_v, acc_deg.at[pl.ds(off, ZR)])
    plsc.subcore_barrier()

    def fire(j, buf, sem):
      pltpu.async_copy(feat_hbm.at[c].at[src_v.at[j]], buf, sem)

    def drain_scatter(j, buf, sem):
      pltpu.make_async_copy(feat_hbm.at[c].at[src_v.at[j]], buf, sem).wait()
      pltpu.sync_copy(buf, acc_sum.at[dst_v.at[j]], add=True)
      pltpu.sync_copy(ones_v, acc_deg.at[dst_v.at[j]], add=True)

    for p in range(NIB):
      blk = s * NIB + p
      pltpu.sync_copy(edge_hbm.at[0].at[blk], src_v)
      pltpu.sync_copy(edge_hbm.at[1].at[blk], dst_v)

      fire(0, rows0_v, sem0)

      def step(j2, _):
        j = 2 * j2
        fire(j + 1, rows1_v, sem1)
        drain_scatter(j, rows0_v, sem0)

        @pl.when(j2 < NCHUNK // 2 - 1)
        def _():
          fire(j + 2, rows0_v, sem0)
        drain_scatter(j + 1, rows1_v, sem1)
        return 0
      lax.fori_loop(0, NCHUNK // 2, step, 0)

    plsc.subcore_barrier()
    for r in range(ROWS_PT // ZR):
      off = s * ROWS_PT + r * ZR
      pltpu.sync_copy(acc_sum.at[pl.ds(off, ZR)], psum_hbm.at[c, pl.ds(off, ZR)])
      pltpu.sync_copy(acc_deg.at[pl.ds(off, ZR)], pdeg_hbm.at[c, pl.ds(off, ZR)])

  return agg(feat2, edge4d)


BLK = 1000


def _combine_body(psum_ref, pdeg_ref, ntype_ref, ws_ref, bs_ref, wn_ref,
                  bn_ref, out_ref):
  s = jnp.concatenate([psum_ref[0], psum_ref[1]], axis=-1)
  deg = 0.5 * (jnp.max(pdeg_ref[0], axis=-1) + jnp.max(pdeg_ref[1], axis=-1))
  h = s / jnp.maximum(deg, 1.0)[:, None]
  self_part = lax.dot_general(
      ntype_ref[...], ws_ref[...], (((1,), (1,)), ((), ())),
      preferred_element_type=jnp.float32)
  neigh_part = lax.dot_general(
      h, wn_ref[...], (((1,), (1,)), ((), ())),
      preferred_element_type=jnp.float32)
  out_ref[...] = self_part + neigh_part + bs_ref[...] + bn_ref[...]


def _combine(psum, pdeg, ntype, W_self, b_self, W_neigh, b_neigh):
  return pl.pallas_call(
      _combine_body,
      grid=(N // BLK,),
      in_specs=[
          pl.BlockSpec((NC, BLK, DH), lambda i: (0, i, 0)),
          pl.BlockSpec((NC, BLK, DEGW), lambda i: (0, i, 0)),
          pl.BlockSpec((BLK, NT), lambda i: (i, 0)),
          pl.BlockSpec((OUT, NT), lambda i: (0, 0)),
          pl.BlockSpec((1, OUT), lambda i: (0, 0)),
          pl.BlockSpec((OUT, D), lambda i: (0, 0)),
          pl.BlockSpec((1, OUT), lambda i: (0, 0)),
      ],
      out_specs=pl.BlockSpec((BLK, OUT), lambda i: (i, 0)),
      out_shape=jax.ShapeDtypeStruct((N, OUT), jnp.float32),
  )(psum, pdeg, ntype, W_self, b_self, W_neigh, b_neigh)


def kernel(act_flag, feat, edge_index, ntype, W_self, b_self, W_neigh,
           b_neigh):
  del act_flag
  feat2 = jnp.stack([feat[:, :DH], feat[:, DH:]])
  edge4d = edge_index.reshape(2, NS * NIB, NCHUNK, CH)
  psum, pdeg = _sc_aggregate(feat2, edge4d)
  return _combine(psum, pdeg, ntype, W_self, b_self.reshape(1, OUT),
                  W_neigh, b_neigh.reshape(1, OUT))


# speedup vs baseline: 11.1031x; 1.0478x over previous
import functools

import jax
import jax.numpy as jnp
from jax import lax
from jax.experimental import pallas as pl
from jax.experimental.pallas import tpu as pltpu
from jax.experimental.pallas import tpu_sc as plsc

N = 10000
E = 320000
D = 128
NT = 8
OUT = 128

NC = 2
NS = 16
DH = D // NC
EPT = E // NS
CH = 125
NIB = 2
NCHUNK = EPT // (CH * NIB)
NPAD = 10240
ROWS_PT = NPAD // NS
ZR = 128
DEGW = 16
L = 16


def _sc_aggregate(feat2, edge4d):
  mesh = plsc.VectorSubcoreMesh(core_axis_name="c", subcore_axis_name="s")

  @functools.partial(
      pl.kernel,
      mesh=mesh,
      out_type=[
          jax.ShapeDtypeStruct((NC, NPAD, DH), jnp.float32),
          jax.ShapeDtypeStruct((NC, NPAD, DEGW), jnp.float32),
      ],
      scratch_types=[
          pltpu.VMEM((NCHUNK, CH), jnp.int32),
          pltpu.VMEM((NCHUNK, CH), jnp.int32),
          pltpu.VMEM((CH, DH), jnp.float32),
          pltpu.VMEM((CH, DH), jnp.float32),
          pltpu.VMEM((CH, DEGW), jnp.float32),
          pltpu.VMEM((ZR, DH), jnp.float32),
          pltpu.VMEM((ZR, DEGW), jnp.float32),
          pltpu.VMEM_SHARED((NPAD, DH), jnp.float32),
          pltpu.VMEM_SHARED((NPAD, DEGW), jnp.float32),
          pltpu.SemaphoreType.DMA,
          pltpu.SemaphoreType.DMA,
          pltpu.SemaphoreType.DMA,
      ],
      compiler_params=pltpu.CompilerParams(use_tc_tiling_on_sc=False),
  )
  def agg(feat_hbm, edge_hbm, psum_hbm, pdeg_hbm,
          src_v, dst_v, rows0_v, rows1_v, ones_v, zb_v, zbd_v,
          acc_sum, acc_deg, sem0, sem1, semo):
    c = lax.axis_index("c")
    s = lax.axis_index("s")

    zeros = jnp.zeros((L,), jnp.float32)
    ones = jnp.full((L,), 1.0, jnp.float32)

    def fill_bufs(i, _):
      for j in range(DH // L):
        zb_v[i, pl.ds(j * L, L)] = zeros
      zbd_v[i] = zeros
      return 0
    lax.fori_loop(0, ZR, fill_bufs, 0)

    def fill_ones(i, _):
      ones_v[i] = ones
      return 0
    lax.fori_loop(0, CH, fill_ones, 0)

    for r in range(ROWS_PT // ZR):
      off = s * ROWS_PT + r * ZR
      pltpu.sync_copy(zb_v, acc_sum.at[pl.ds(off, ZR)])
      pltpu.sync_copy(zbd_v, acc_deg.at[pl.ds(off, ZR)])
    plsc.subcore_barrier()

    def fire(j, buf, sem):
      pltpu.async_copy(feat_hbm.at[c].at[src_v.at[j]], buf, sem)

    for p in range(NIB):
      blk = s * NIB + p
      pltpu.sync_copy(edge_hbm.at[0].at[blk], src_v)
      pltpu.sync_copy(edge_hbm.at[1].at[blk], dst_v)

      fire(0, rows0_v, sem0)

      def step(j2, _, p=p):
        j = 2 * j2
        fire(j + 1, rows1_v, sem1)
        pltpu.make_async_copy(feat_hbm.at[c].at[src_v.at[j]], rows0_v,
                              sem0).wait()
        pltpu.sync_copy(rows0_v, acc_sum.at[dst_v.at[j]], add=True)

        @pl.when(c == p)
        def _():
          pltpu.async_copy(ones_v, acc_deg.at[dst_v.at[j]], semo, add=True)

        @pl.when(j2 < NCHUNK // 2 - 1)
        def _():
          fire(j + 2, rows0_v, sem0)
        pltpu.make_async_copy(feat_hbm.at[c].at[src_v.at[j + 1]], rows1_v,
                              sem1).wait()
        pltpu.sync_copy(rows1_v, acc_sum.at[dst_v.at[j + 1]], add=True)

        @pl.when(c == p)
        def _():
          pltpu.async_copy(ones_v, acc_deg.at[dst_v.at[j + 1]], semo, add=True)
        return 0
      lax.fori_loop(0, NCHUNK // 2, step, 0)

      @pl.when(c == p)
      def _():
        def drain(j, _):
          pltpu.make_async_copy(ones_v, acc_deg.at[dst_v.at[j]], semo).wait()
          return 0
        lax.fori_loop(0, NCHUNK, drain, 0)

    plsc.subcore_barrier()
    for r in range(ROWS_PT // ZR):
      off = s * ROWS_PT + r * ZR
      pltpu.sync_copy(acc_sum.at[pl.ds(off, ZR)], psum_hbm.at[c, pl.ds(off, ZR)])
      pltpu.sync_copy(acc_deg.at[pl.ds(off, ZR)], pdeg_hbm.at[c, pl.ds(off, ZR)])

  return agg(feat2, edge4d)


BLK = 1000


def _combine_body(psum_ref, pdeg_ref, ntype_ref, ws_ref, bs_ref, wn_ref,
                  bn_ref, out_ref):
  s = jnp.concatenate([psum_ref[0], psum_ref[1]], axis=-1)
  deg = jnp.max(pdeg_ref[0], axis=-1) + jnp.max(pdeg_ref[1], axis=-1)
  h = s / jnp.maximum(deg, 1.0)[:, None]
  self_part = lax.dot_general(
      ntype_ref[...], ws_ref[...], (((1,), (1,)), ((), ())),
      preferred_element_type=jnp.float32)
  neigh_part = lax.dot_general(
      h, wn_ref[...], (((1,), (1,)), ((), ())),
      preferred_element_type=jnp.float32)
  out_ref[...] = self_part + neigh_part + bs_ref[...] + bn_ref[...]


def _combine(psum, pdeg, ntype, W_self, b_self, W_neigh, b_neigh):
  return pl.pallas_call(
      _combine_body,
      grid=(N // BLK,),
      in_specs=[
          pl.BlockSpec((NC, BLK, DH), lambda i: (0, i, 0)),
          pl.BlockSpec((NC, BLK, DEGW), lambda i: (0, i, 0)),
          pl.BlockSpec((BLK, NT), lambda i: (i, 0)),
          pl.BlockSpec((OUT, NT), lambda i: (0, 0)),
          pl.BlockSpec((1, OUT), lambda i: (0, 0)),
          pl.BlockSpec((OUT, D), lambda i: (0, 0)),
          pl.BlockSpec((1, OUT), lambda i: (0, 0)),
      ],
      out_specs=pl.BlockSpec((BLK, OUT), lambda i: (i, 0)),
      out_shape=jax.ShapeDtypeStruct((N, OUT), jnp.float32),
  )(psum, pdeg, ntype, W_self, b_self, W_neigh, b_neigh)


def kernel(act_flag, feat, edge_index, ntype, W_self, b_self, W_neigh,
           b_neigh):
  del act_flag
  feat2 = jnp.stack([feat[:, :DH], feat[:, DH:]])
  edge4d = edge_index.reshape(2, NS * NIB, NCHUNK, CH)
  psum, pdeg = _sc_aggregate(feat2, edge4d)
  return _combine(psum, pdeg, ntype, W_self, b_self.reshape(1, OUT),
                  W_neigh, b_neigh.reshape(1, OUT))


# speedup vs baseline: 12.2064x; 1.0994x over previous
import functools

import jax
import jax.numpy as jnp
from jax import lax
from jax.experimental import pallas as pl
from jax.experimental.pallas import tpu as pltpu
from jax.experimental.pallas import tpu_sc as plsc

N = 10000
E = 320000
D = 128
NT = 8
OUT = 128

NC = 2
NS = 16
DH = D // NC
EPT = E // NS
CH = 125
NIB = 2
NCHUNK = EPT // (CH * NIB)
NPAD = 10240
ROWS_PT = NPAD // NS
ZR = 128
DEGW = 16
L = 16


def _sc_aggregate(feat2, edge4d):
  mesh = plsc.VectorSubcoreMesh(core_axis_name="c", subcore_axis_name="s")

  @functools.partial(
      pl.kernel,
      mesh=mesh,
      out_type=[
          jax.ShapeDtypeStruct((NPAD, D), jnp.float32),
          jax.ShapeDtypeStruct((NPAD, NC * DEGW), jnp.float32),
      ],
      scratch_types=[
          pltpu.VMEM((NCHUNK, CH), jnp.int32),
          pltpu.VMEM((NCHUNK, CH), jnp.int32),
          pltpu.VMEM((CH, DH), jnp.float32),
          pltpu.VMEM((CH, DH), jnp.float32),
          pltpu.VMEM((CH, DEGW), jnp.float32),
          pltpu.VMEM((ZR, DH), jnp.float32),
          pltpu.VMEM((ZR, DEGW), jnp.float32),
          pltpu.VMEM_SHARED((NPAD, DH), jnp.float32),
          pltpu.VMEM_SHARED((NPAD, DEGW), jnp.float32),
          pltpu.SemaphoreType.DMA,
          pltpu.SemaphoreType.DMA,
          pltpu.SemaphoreType.DMA,
      ],
      compiler_params=pltpu.CompilerParams(use_tc_tiling_on_sc=False),
  )
  def agg(feat_hbm, edge_hbm, psum_hbm, pdeg_hbm,
          src_v, dst_v, rows0_v, rows1_v, ones_v, zb_v, zbd_v,
          acc_sum, acc_deg, sem0, sem1, semo):
    c = lax.axis_index("c")
    s = lax.axis_index("s")

    zeros = jnp.zeros((L,), jnp.float32)
    ones = jnp.full((L,), 1.0, jnp.float32)

    def fill_bufs(i, _):
      for j in range(DH // L):
        zb_v[i, pl.ds(j * L, L)] = zeros
      zbd_v[i] = zeros
      return 0
    lax.fori_loop(0, ZR, fill_bufs, 0)

    def fill_ones(i, _):
      ones_v[i] = ones
      return 0
    lax.fori_loop(0, CH, fill_ones, 0)

    for r in range(ROWS_PT // ZR):
      off = s * ROWS_PT + r * ZR
      pltpu.sync_copy(zb_v, acc_sum.at[pl.ds(off, ZR)])
      pltpu.sync_copy(zbd_v, acc_deg.at[pl.ds(off, ZR)])
    plsc.subcore_barrier()

    def fire(j, buf, sem):
      pltpu.async_copy(feat_hbm.at[c].at[src_v.at[j]], buf, sem)

    for p in range(NIB):
      blk = s * NIB + p
      pltpu.sync_copy(edge_hbm.at[0].at[blk], src_v)
      pltpu.sync_copy(edge_hbm.at[1].at[blk], dst_v)

      fire(0, rows0_v, sem0)

      def step(j2, _, p=p):
        j = 2 * j2
        fire(j + 1, rows1_v, sem1)
        pltpu.make_async_copy(feat_hbm.at[c].at[src_v.at[j]], rows0_v,
                              sem0).wait()
        pltpu.sync_copy(rows0_v, acc_sum.at[dst_v.at[j]], add=True)

        @pl.when(c == p)
        def _():
          pltpu.async_copy(ones_v, acc_deg.at[dst_v.at[j]], semo, add=True)

        @pl.when(j2 < NCHUNK // 2 - 1)
        def _():
          fire(j + 2, rows0_v, sem0)
        pltpu.make_async_copy(feat_hbm.at[c].at[src_v.at[j + 1]], rows1_v,
                              sem1).wait()
        pltpu.sync_copy(rows1_v, acc_sum.at[dst_v.at[j + 1]], add=True)

        @pl.when(c == p)
        def _():
          pltpu.async_copy(ones_v, acc_deg.at[dst_v.at[j + 1]], semo, add=True)
        return 0
      lax.fori_loop(0, NCHUNK // 2, step, 0)

      @pl.when(c == p)
      def _():
        def drain(j, _):
          pltpu.make_async_copy(ones_v, acc_deg.at[dst_v.at[j]], semo).wait()
          return 0
        lax.fori_loop(0, NCHUNK, drain, 0)

    plsc.subcore_barrier()
    for r in range(ROWS_PT // ZR):
      off = s * ROWS_PT + r * ZR
      pltpu.sync_copy(acc_sum.at[pl.ds(off, ZR)],
                      psum_hbm.at[pl.ds(off, ZR), pl.ds(c * DH, DH)])
      pltpu.sync_copy(acc_deg.at[pl.ds(off, ZR)],
                      pdeg_hbm.at[pl.ds(off, ZR), pl.ds(c * DEGW, DEGW)])

  return agg(feat2, edge4d)


BLK = 1000


def _combine_body(psum_ref, pdeg_ref, ntype_ref, ws_ref, bs_ref, wn_ref,
                  bn_ref, out_ref):
  s = psum_ref[...]
  deg = (jnp.max(pdeg_ref[:, :DEGW], axis=-1)
         + jnp.max(pdeg_ref[:, DEGW:], axis=-1))
  h = s / jnp.maximum(deg, 1.0)[:, None]
  self_part = lax.dot_general(
      ntype_ref[...], ws_ref[...], (((1,), (1,)), ((), ())),
      preferred_element_type=jnp.float32)
  neigh_part = lax.dot_general(
      h, wn_ref[...], (((1,), (1,)), ((), ())),
      preferred_element_type=jnp.float32)
  out_ref[...] = self_part + neigh_part + bs_ref[...] + bn_ref[...]


def _combine(psum, pdeg, ntype, W_self, b_self, W_neigh, b_neigh):
  return pl.pallas_call(
      _combine_body,
      grid=(N // BLK,),
      in_specs=[
          pl.BlockSpec((BLK, D), lambda i: (i, 0)),
          pl.BlockSpec((BLK, NC * DEGW), lambda i: (i, 0)),
          pl.BlockSpec((BLK, NT), lambda i: (i, 0)),
          pl.BlockSpec((OUT, NT), lambda i: (0, 0)),
          pl.BlockSpec((1, OUT), lambda i: (0, 0)),
          pl.BlockSpec((OUT, D), lambda i: (0, 0)),
          pl.BlockSpec((1, OUT), lambda i: (0, 0)),
      ],
      out_specs=pl.BlockSpec((BLK, OUT), lambda i: (i, 0)),
      out_shape=jax.ShapeDtypeStruct((N, OUT), jnp.float32),
  )(psum, pdeg, ntype, W_self, b_self, W_neigh, b_neigh)


def kernel(act_flag, feat, edge_index, ntype, W_self, b_self, W_neigh,
           b_neigh):
  del act_flag
  feat2 = jnp.stack([feat[:, :DH], feat[:, DH:]])
  edge4d = edge_index.reshape(2, NS * NIB, NCHUNK, CH)
  psum, pdeg = _sc_aggregate(feat2, edge4d)
  return _combine(psum, pdeg, ntype, W_self, b_self.reshape(1, OUT),
                  W_neigh, b_neigh.reshape(1, OUT))
